# baseline (speedup 1.0000x reference)
import functools

import jax
import jax.numpy as jnp
from jax.experimental import pallas as pl
from jax.experimental.pallas import tpu as pltpu

_SQRT1_2 = 0.7071067811865476


def _gelu(x):
    return 0.5 * x * (1.0 + jax.lax.erf(x * _SQRT1_2))


def _kl_term(mu1, mu2, ls1, ls2):
    return -0.5 + ls2 - ls1 + 0.5 * (
        jnp.exp(2.0 * (ls1 - ls2)) + (mu1 - mu2) ** 2 * jnp.exp(-2.0 * ls2))


def _dgt(w, v):
    return jax.lax.dot_general(w, v, (((0,), (0,)), ((), ())),
                               preferred_element_type=jnp.float32)


def _fwd_kernel(full_ref, part_ref, epst_ref,
                e0, e1, e2, e3, p0, p1, p2, p3, zw, rp_ref, bp_ref,
                x_ref, z_ref, kl_ref, klp_ref, klq_ref,
                z_s, kl_s, klp_s, klq_s, *, zd):
    c, hw = full_ref.shape
    b = pl.program_id(0)
    full = full_ref[...]
    fvec = jnp.mean(full, axis=1, keepdims=True)
    pvec = jnp.mean(part_ref[...], axis=1, keepdims=True)

    v = fvec
    for w_ref, boff in ((e0, 0), (e1, c), (e2, 2 * c)):
        v = _dgt(w_ref[...], _gelu(v)) + bp_ref[boff:boff + c]
    ev = _dgt(e3[...], _gelu(v)) + bp_ref[3 * c:3 * c + 2 * zd]

    pb0 = 3 * c + 2 * zd
    u = pvec
    for w_ref, boff in ((p0, pb0), (p1, pb0 + c), (p2, pb0 + 2 * c)):
        u = _dgt(w_ref[...], _gelu(u)) + bp_ref[boff:boff + c]
    po = _dgt(p3[...], _gelu(u)) + bp_ref[pb0 + 3 * c:pb0 + 4 * c + 2 * zd]

    qm, qv = ev[0:zd], ev[zd:2 * zd]
    pm, pvr = po[0:zd], po[zd:2 * zd]
    xpp = po[2 * zd:]
    et = epst_ref[...]
    lane_e = jax.lax.broadcasted_iota(jnp.int32, et.shape, 1)
    eps = jnp.sum(jnp.where(lane_e == b, et, 0.0), axis=1, keepdims=True)

    z = jnp.exp(qv) * eps + qm
    zb0 = 7 * c + 4 * zd
    xs = xpp + _dgt(zw[...], z) + bp_ref[zb0:zb0 + c]

    lane = jax.lax.broadcasted_iota(jnp.int32, z_s.shape, 1)
    z_s[...] = jnp.where(lane == b, z, z_s[...])
    kl_s[...] = jnp.where(lane == b, _kl_term(qm, pm, qv, pvr), kl_s[...])
    klq_s[...] = jnp.where(lane == b, _kl_term(qm, 0.0, qv, 0.0), klq_s[...])
    klp_s[...] = jnp.where(lane == b, _kl_term(pm, 0.0, pvr, 0.0), klp_s[...])

    @pl.when(b == pl.num_programs(0) - 1)
    def _():
        z_ref[...] = z_s[...]
        kl_ref[...] = kl_s[...]
        klp_ref[...] = klp_s[...]
        klq_ref[...] = klq_s[...]

    xin = full + xs
    rb0 = 8 * c + 4 * zd
    hh = xin
    for i in range(4):
        g = _gelu(hh).astype(jnp.bfloat16)
        hh = _dgt(rp_ref[:, i * c:(i + 1) * c], g) + bp_ref[rb0 + i * c:
                                                            rb0 + (i + 1) * c]
    x_ref[...] = xin + hh


def kernel(full_acts, part_acts, eps,
           enc0_w, enc0_b, enc1_w, enc1_b, enc2_w, enc2_b, enc3_w, enc3_b,
           prior0_w, prior0_b, prior1_w, prior1_b, prior2_w, prior2_b,
           prior3_w, prior3_b,
           res0_w, res0_b, res1_w, res1_b, res2_w, res2_b, res3_w, res3_b,
           zp_w, zp_b):
    B, C, H, W = full_acts.shape
    HW = H * W
    zd = eps.shape[1]

    full2 = full_acts.reshape(B * C, HW)
    part2 = part_acts.reshape(B * C, HW)
    epst = eps.T
    rpack = jnp.concatenate([res0_w, res1_w, res2_w, res3_w],
                            axis=1).astype(jnp.bfloat16)
    bpackt = jnp.concatenate(
        [enc0_b, enc1_b, enc2_b, enc3_b, prior0_b, prior1_b, prior2_b,
         prior3_b, zp_b, res0_b, res1_b, res2_b, res3_b], axis=1).T

    whole = lambda a: pl.BlockSpec(a.shape, lambda b: (0,) * a.ndim)
    smt = jax.ShapeDtypeStruct((zd, B), jnp.float32)
    sm_spec = pl.BlockSpec((zd, B), lambda b: (0, 0))

    xd, zt, klt, klpt, klqt = pl.pallas_call(
        functools.partial(_fwd_kernel, zd=zd),
        grid=(B,),
        in_specs=[pl.BlockSpec((C, HW), lambda b: (b, 0)),
                  pl.BlockSpec((C, HW), lambda b: (b, 0)),
                  whole(epst),
                  whole(enc0_w), whole(enc1_w), whole(enc2_w), whole(enc3_w),
                  whole(prior0_w), whole(prior1_w), whole(prior2_w),
                  whole(prior3_w), whole(zp_w), whole(rpack), whole(bpackt)],
        out_specs=(pl.BlockSpec((C, HW), lambda b: (b, 0)),
                   sm_spec, sm_spec, sm_spec, sm_spec),
        out_shape=(jax.ShapeDtypeStruct((B * C, HW), jnp.float32),
                   smt, smt, smt, smt),
        scratch_shapes=[pltpu.VMEM((zd, B), jnp.float32) for _ in range(4)],
        compiler_params=pltpu.CompilerParams(
            dimension_semantics=("arbitrary",),
            vmem_limit_bytes=48 * 1024 * 1024),
    )(full2, part2, epst, enc0_w, enc1_w, enc2_w, enc3_w,
      prior0_w, prior1_w, prior2_w, prior3_w, zp_w, rpack, bpackt)

    x = xd.reshape(B, C, H, W)
    to4 = lambda t: t.T[:, :, None, None]
    return to4(zt), x, to4(klt), to4(klpt), to4(klqt)


# speedup vs baseline: 1.7779x; 1.7779x over previous
import functools

import jax
import jax.numpy as jnp
from jax.experimental import pallas as pl
from jax.experimental.pallas import tpu as pltpu

_SQRT1_2 = 0.7071067811865476


def _gelu(x):
    return 0.5 * x * (1.0 + jax.lax.erf(x * _SQRT1_2))


def _kl_term(mu1, mu2, ls1, ls2):
    return -0.5 + ls2 - ls1 + 0.5 * (
        jnp.exp(2.0 * (ls1 - ls2)) + (mu1 - mu2) ** 2 * jnp.exp(-2.0 * ls2))


def _dgt(w, v, prec=jnp.float32):
    return jax.lax.dot_general(w, v, (((0,), (0,)), ((), ())),
                               preferred_element_type=prec)


def _fwd_kernel(full_ref, part_ref, eps_ref,
                e0, e1, e2, e3, p0, p1, p2, p3, zw, rp_ref, bp_ref,
                z_ref, x_ref, kl_ref, klp_ref, klq_ref, *, zd):
    c, hw = full_ref.shape[1:]
    full = full_ref[0]
    fvec = jnp.mean(full, axis=1, keepdims=True)
    pvec = jnp.mean(part_ref[0], axis=1, keepdims=True)

    v = fvec
    for w_ref, boff in ((e0, 0), (e1, c), (e2, 2 * c)):
        v = _dgt(w_ref[...], _gelu(v)) + bp_ref[boff:boff + c]
    ev = _dgt(e3[...], _gelu(v)) + bp_ref[3 * c:3 * c + 2 * zd]

    pb0 = 3 * c + 2 * zd
    u = pvec
    for w_ref, boff in ((p0, pb0), (p1, pb0 + c), (p2, pb0 + 2 * c)):
        u = _dgt(w_ref[...], _gelu(u)) + bp_ref[boff:boff + c]
    po = _dgt(p3[...], _gelu(u)) + bp_ref[pb0 + 3 * c:pb0 + 4 * c + 2 * zd]

    qm, qv = ev[0:zd], ev[zd:2 * zd]
    pm, pvr = po[0:zd], po[zd:2 * zd]
    xpp = po[2 * zd:]
    eps = eps_ref[0]

    z = jnp.exp(qv) * eps + qm
    zb0 = 7 * c + 4 * zd
    xs = xpp + _dgt(zw[...], z) + bp_ref[zb0:zb0 + c]

    kl = _kl_term(qm, pm, qv, pvr)
    klq = _kl_term(qm, 0.0, qv, 0.0)
    klp = _kl_term(pm, 0.0, pvr, 0.0)
    z_ref[0] = z.reshape(zd, 1, 1)
    kl_ref[0] = kl.reshape(zd, 1, 1)
    klq_ref[0] = klq.reshape(zd, 1, 1)
    klp_ref[0] = klp.reshape(zd, 1, 1)

    xin = full + xs
    rb0 = 8 * c + 4 * zd
    hh = xin
    for i in range(4):
        g = _gelu(hh).astype(jnp.bfloat16)
        hh = _dgt(rp_ref[:, i * c:(i + 1) * c], g) + bp_ref[rb0 + i * c:
                                                            rb0 + (i + 1) * c]
    x_ref[0] = xin + hh


def kernel(full_acts, part_acts, eps,
           enc0_w, enc0_b, enc1_w, enc1_b, enc2_w, enc2_b, enc3_w, enc3_b,
           prior0_w, prior0_b, prior1_w, prior1_b, prior2_w, prior2_b,
           prior3_w, prior3_b,
           res0_w, res0_b, res1_w, res1_b, res2_w, res2_b, res3_w, res3_b,
           zp_w, zp_b):
    B, C, H, W = full_acts.shape
    HW = H * W
    zd = eps.shape[1]

    full2 = full_acts.reshape(B, C, HW)
    part2 = part_acts.reshape(B, C, HW)
    eps3 = eps[:, :, None]
    rpack = jnp.concatenate([res0_w, res1_w, res2_w, res3_w],
                            axis=1).astype(jnp.bfloat16)
    bpackt = jnp.concatenate(
        [enc0_b, enc1_b, enc2_b, enc3_b, prior0_b, prior1_b, prior2_b,
         prior3_b, zp_b, res0_b, res1_b, res2_b, res3_b], axis=1).T

    whole = lambda a: pl.BlockSpec(a.shape, lambda b: (0,) * a.ndim)
    small = jax.ShapeDtypeStruct((B, zd, 1, 1), jnp.float32)
    small_spec = pl.BlockSpec((1, zd, 1, 1), lambda b: (b, 0, 0, 0))

    def run(f2, p2, e3_, ew0, ew1, ew2, ew3, pw0, pw1, pw2, pw3, zw, rp, bp):
        nloc = f2.shape[0]
        sm = jax.ShapeDtypeStruct((nloc, zd, 1, 1), jnp.float32)
        return pl.pallas_call(
            functools.partial(_fwd_kernel, zd=zd),
            grid=(nloc,),
            in_specs=[pl.BlockSpec((1, C, HW), lambda b: (b, 0, 0)),
                      pl.BlockSpec((1, C, HW), lambda b: (b, 0, 0)),
                      pl.BlockSpec((1, zd, 1), lambda b: (b, 0, 0)),
                      whole(ew0), whole(ew1), whole(ew2), whole(ew3),
                      whole(pw0), whole(pw1), whole(pw2), whole(pw3),
                      whole(zw), whole(rp), whole(bp)],
            out_specs=(small_spec,
                       pl.BlockSpec((1, C, HW), lambda b: (b, 0, 0)),
                       small_spec, small_spec, small_spec),
            out_shape=(sm,
                       jax.ShapeDtypeStruct((nloc, C, HW), jnp.float32),
                       sm, sm, sm),
            compiler_params=pltpu.CompilerParams(
                dimension_semantics=("parallel",),
                vmem_limit_bytes=48 * 1024 * 1024),
        )(f2, p2, e3_, ew0, ew1, ew2, ew3, pw0, pw1, pw2, pw3, zw, rp, bp)

    z4, xd, kl4, klp4, klq4 = run(
        full2, part2, eps3, enc0_w, enc1_w, enc2_w, enc3_w,
        prior0_w, prior1_w, prior2_w, prior3_w, zp_w, rpack, bpackt)

    x = xd.reshape(B, C, H, W)
    return z4, x, kl4, klp4, klq4


# speedup vs baseline: 2.0327x; 1.1433x over previous
import functools

import jax
import jax.numpy as jnp
from jax.experimental import pallas as pl
from jax.experimental.pallas import tpu as pltpu

_SQRT1_2 = 0.7071067811865476


def _gelu(x):
    return 0.5 * x * (1.0 + jax.lax.erf(x * _SQRT1_2))


def _kl_term(mu1, mu2, ls1, ls2):
    return -0.5 + ls2 - ls1 + 0.5 * (
        jnp.exp(2.0 * (ls1 - ls2)) + (mu1 - mu2) ** 2 * jnp.exp(-2.0 * ls2))


def _dgt(w, v, prec=jnp.float32):
    return jax.lax.dot_general(w, v, (((0,), (0,)), ((), ())),
                               preferred_element_type=prec)


def _fwd_kernel(full_ref, part_ref, eps_ref,
                e0, e1, e2, e3, p0, p1, p2, p3, zw, rp_ref, bp_ref,
                z_ref, x_ref, kl_ref, klp_ref, klq_ref, *, zd, nb):
    c, hw = full_ref.shape[1:]
    fvec = jnp.concatenate(
        [jnp.mean(full_ref[i], axis=1, keepdims=True) for i in range(nb)],
        axis=1)
    pvec = jnp.concatenate(
        [jnp.mean(part_ref[i], axis=1, keepdims=True) for i in range(nb)],
        axis=1)

    v = fvec
    for w_ref, boff in ((e0, 0), (e1, c), (e2, 2 * c)):
        v = _dgt(w_ref[...], _gelu(v)) + bp_ref[boff:boff + c]
    ev = _dgt(e3[...], _gelu(v)) + bp_ref[3 * c:3 * c + 2 * zd]

    pb0 = 3 * c + 2 * zd
    u = pvec
    for w_ref, boff in ((p0, pb0), (p1, pb0 + c), (p2, pb0 + 2 * c)):
        u = _dgt(w_ref[...], _gelu(u)) + bp_ref[boff:boff + c]
    po = _dgt(p3[...], _gelu(u)) + bp_ref[pb0 + 3 * c:pb0 + 4 * c + 2 * zd]

    qm, qv = ev[0:zd], ev[zd:2 * zd]
    pm, pvr = po[0:zd], po[zd:2 * zd]
    xpp = po[2 * zd:]
    eps = jnp.transpose(eps_ref[...][:, :, 0])

    z = jnp.exp(qv) * eps + qm
    zb0 = 7 * c + 4 * zd
    xs = xpp + _dgt(zw[...], z) + bp_ref[zb0:zb0 + c]

    kl = _kl_term(qm, pm, qv, pvr)
    klq = _kl_term(qm, 0.0, qv, 0.0)
    klp = _kl_term(pm, 0.0, pvr, 0.0)
    z_ref[...] = jnp.transpose(z).reshape(nb, zd, 1, 1)
    kl_ref[...] = jnp.transpose(kl).reshape(nb, zd, 1, 1)
    klq_ref[...] = jnp.transpose(klq).reshape(nb, zd, 1, 1)
    klp_ref[...] = jnp.transpose(klp).reshape(nb, zd, 1, 1)

    rb0 = 8 * c + 4 * zd
    for i in range(nb):
        xin = full_ref[i] + xs[:, i:i + 1]
        hh = xin
        for l in range(4):
            g = _gelu(hh).astype(jnp.bfloat16)
            hh = _dgt(rp_ref[:, l * c:(l + 1) * c], g) + bp_ref[
                rb0 + l * c:rb0 + (l + 1) * c]
        x_ref[i] = xin + hh


def kernel(full_acts, part_acts, eps,
           enc0_w, enc0_b, enc1_w, enc1_b, enc2_w, enc2_b, enc3_w, enc3_b,
           prior0_w, prior0_b, prior1_w, prior1_b, prior2_w, prior2_b,
           prior3_w, prior3_b,
           res0_w, res0_b, res1_w, res1_b, res2_w, res2_b, res3_w, res3_b,
           zp_w, zp_b):
    B, C, H, W = full_acts.shape
    HW = H * W
    zd = eps.shape[1]

    full2 = full_acts.reshape(B, C, HW)
    part2 = part_acts.reshape(B, C, HW)
    eps3 = eps[:, :, None]
    rpack = jnp.concatenate([res0_w, res1_w, res2_w, res3_w],
                            axis=1).astype(jnp.bfloat16)
    bpackt = jnp.concatenate(
        [enc0_b, enc1_b, enc2_b, enc3_b, prior0_b, prior1_b, prior2_b,
         prior3_b, zp_b, res0_b, res1_b, res2_b, res3_b], axis=1).T

    whole = lambda a: pl.BlockSpec(a.shape, lambda b: (0,) * a.ndim)
    nb = 4 if B % 4 == 0 else 1
    small_spec = pl.BlockSpec((nb, zd, 1, 1), lambda b: (b, 0, 0, 0))

    def run(f2, p2, e3_, ew0, ew1, ew2, ew3, pw0, pw1, pw2, pw3, zw, rp, bp):
        nloc = f2.shape[0]
        sm = jax.ShapeDtypeStruct((nloc, zd, 1, 1), jnp.float32)
        return pl.pallas_call(
            functools.partial(_fwd_kernel, zd=zd, nb=nb),
            grid=(nloc // nb,),
            in_specs=[pl.BlockSpec((nb, C, HW), lambda b: (b, 0, 0)),
                      pl.BlockSpec((nb, C, HW), lambda b: (b, 0, 0)),
                      pl.BlockSpec((nb, zd, 1), lambda b: (b, 0, 0)),
                      whole(ew0), whole(ew1), whole(ew2), whole(ew3),
                      whole(pw0), whole(pw1), whole(pw2), whole(pw3),
                      whole(zw), whole(rp), whole(bp)],
            out_specs=(small_spec,
                       pl.BlockSpec((nb, C, HW), lambda b: (b, 0, 0)),
                       small_spec, small_spec, small_spec),
            out_shape=(sm,
                       jax.ShapeDtypeStruct((nloc, C, HW), jnp.float32),
                       sm, sm, sm),
            compiler_params=pltpu.CompilerParams(
                dimension_semantics=("parallel",),
                vmem_limit_bytes=56 * 1024 * 1024),
        )(f2, p2, e3_, ew0, ew1, ew2, ew3, pw0, pw1, pw2, pw3, zw, rp, bp)

    z4, xd, kl4, klp4, klq4 = run(
        full2, part2, eps3, enc0_w, enc1_w, enc2_w, enc3_w,
        prior0_w, prior1_w, prior2_w, prior3_w, zp_w, rpack, bpackt)

    x = xd.reshape(B, C, H, W)
    return z4, x, kl4, klp4, klq4


# speedup vs baseline: 2.1462x; 1.0558x over previous
import functools

import jax
import jax.numpy as jnp
from jax.experimental import pallas as pl
from jax.experimental.pallas import tpu as pltpu

_SQRT1_2 = 0.7071067811865476


def _gelu(x):
    return 0.5 * x * (1.0 + jax.lax.erf(x * _SQRT1_2))


def _kl_term(mu1, mu2, ls1, ls2):
    return -0.5 + ls2 - ls1 + 0.5 * (
        jnp.exp(2.0 * (ls1 - ls2)) + (mu1 - mu2) ** 2 * jnp.exp(-2.0 * ls2))


def _dgt(w, v, prec=jnp.float32):
    return jax.lax.dot_general(w, v, (((0,), (0,)), ((), ())),
                               preferred_element_type=prec)


def _fwd_kernel(full_ref, part_ref, eps_ref,
                e0, e1, e2, e3, p0, p1, p2, p3, zw, rp_ref, bp_ref,
                z_ref, x_ref, kl_ref, klp_ref, klq_ref, *, zd, nb):
    c, hw = full_ref.shape[1:]
    fvec = jnp.concatenate(
        [jnp.mean(full_ref[i], axis=1, keepdims=True) for i in range(nb)],
        axis=1)
    pvec = jnp.concatenate(
        [jnp.mean(part_ref[i], axis=1, keepdims=True) for i in range(nb)],
        axis=1)

    v = fvec
    for w_ref, boff in ((e0, 0), (e1, c), (e2, 2 * c)):
        v = _dgt(w_ref[...], _gelu(v)) + bp_ref[boff:boff + c]
    ev = _dgt(e3[...], _gelu(v)) + bp_ref[3 * c:3 * c + 2 * zd]

    pb0 = 3 * c + 2 * zd
    u = pvec
    for w_ref, boff in ((p0, pb0), (p1, pb0 + c), (p2, pb0 + 2 * c)):
        u = _dgt(w_ref[...], _gelu(u)) + bp_ref[boff:boff + c]
    po = _dgt(p3[...], _gelu(u)) + bp_ref[pb0 + 3 * c:pb0 + 4 * c + 2 * zd]

    qm, qv = ev[0:zd], ev[zd:2 * zd]
    pm, pvr = po[0:zd], po[zd:2 * zd]
    xpp = po[2 * zd:]
    eps = jnp.transpose(eps_ref[...][:, :, 0])

    z = jnp.exp(qv) * eps + qm
    zb0 = 7 * c + 4 * zd
    xs = xpp + _dgt(zw[...], z) + bp_ref[zb0:zb0 + c]

    kl = _kl_term(qm, pm, qv, pvr)
    klq = _kl_term(qm, 0.0, qv, 0.0)
    klp = _kl_term(pm, 0.0, pvr, 0.0)
    z_ref[...] = jnp.transpose(z).reshape(nb, zd, 1, 1)
    kl_ref[...] = jnp.transpose(kl).reshape(nb, zd, 1, 1)
    klq_ref[...] = jnp.transpose(klq).reshape(nb, zd, 1, 1)
    klp_ref[...] = jnp.transpose(klp).reshape(nb, zd, 1, 1)

    rb0 = 8 * c + 4 * zd
    for i in range(nb):
        xin = full_ref[i] + xs[:, i:i + 1]
        hh = xin
        for l in range(4):
            hb = hh.astype(jnp.bfloat16)
            t = jax.lax.erf(hh * _SQRT1_2).astype(jnp.bfloat16)
            g = hb * (jnp.bfloat16(1.0) + t)
            hh = _dgt(rp_ref[:, l * c:(l + 1) * c], g) + bp_ref[
                rb0 + l * c:rb0 + (l + 1) * c]
        x_ref[i] = xin + hh


def kernel(full_acts, part_acts, eps,
           enc0_w, enc0_b, enc1_w, enc1_b, enc2_w, enc2_b, enc3_w, enc3_b,
           prior0_w, prior0_b, prior1_w, prior1_b, prior2_w, prior2_b,
           prior3_w, prior3_b,
           res0_w, res0_b, res1_w, res1_b, res2_w, res2_b, res3_w, res3_b,
           zp_w, zp_b):
    B, C, H, W = full_acts.shape
    HW = H * W
    zd = eps.shape[1]

    full2 = full_acts.reshape(B, C, HW)
    part2 = part_acts.reshape(B, C, HW)
    eps3 = eps[:, :, None]
    rpack = (0.5 * jnp.concatenate([res0_w, res1_w, res2_w, res3_w],
                                   axis=1)).astype(jnp.bfloat16)
    bpackt = jnp.concatenate(
        [enc0_b, enc1_b, enc2_b, enc3_b, prior0_b, prior1_b, prior2_b,
         prior3_b, zp_b, res0_b, res1_b, res2_b, res3_b], axis=1).T

    whole = lambda a: pl.BlockSpec(a.shape, lambda b: (0,) * a.ndim)
    nb = 4 if B % 4 == 0 else 1
    small_spec = pl.BlockSpec((nb, zd, 1, 1), lambda b: (b, 0, 0, 0))

    def run(f2, p2, e3_, ew0, ew1, ew2, ew3, pw0, pw1, pw2, pw3, zw, rp, bp):
        nloc = f2.shape[0]
        sm = jax.ShapeDtypeStruct((nloc, zd, 1, 1), jnp.float32)
        return pl.pallas_call(
            functools.partial(_fwd_kernel, zd=zd, nb=nb),
            grid=(nloc // nb,),
            in_specs=[pl.BlockSpec((nb, C, HW), lambda b: (b, 0, 0)),
                      pl.BlockSpec((nb, C, HW), lambda b: (b, 0, 0)),
                      pl.BlockSpec((nb, zd, 1), lambda b: (b, 0, 0)),
                      whole(ew0), whole(ew1), whole(ew2), whole(ew3),
                      whole(pw0), whole(pw1), whole(pw2), whole(pw3),
                      whole(zw), whole(rp), whole(bp)],
            out_specs=(small_spec,
                       pl.BlockSpec((nb, C, HW), lambda b: (b, 0, 0)),
                       small_spec, small_spec, small_spec),
            out_shape=(sm,
                       jax.ShapeDtypeStruct((nloc, C, HW), jnp.float32),
                       sm, sm, sm),
            compiler_params=pltpu.CompilerParams(
                dimension_semantics=("parallel",),
                vmem_limit_bytes=56 * 1024 * 1024),
        )(f2, p2, e3_, ew0, ew1, ew2, ew3, pw0, pw1, pw2, pw3, zw, rp, bp)

    z4, xd, kl4, klp4, klq4 = run(
        full2, part2, eps3, enc0_w, enc1_w, enc2_w, enc3_w,
        prior0_w, prior1_w, prior2_w, prior3_w, zp_w, rpack, bpackt)

    x = xd.reshape(B, C, H, W)
    return z4, x, kl4, klp4, klq4


# speedup vs baseline: 2.1517x; 1.0026x over previous
import functools

import jax
import jax.numpy as jnp
from jax.experimental import pallas as pl
from jax.experimental.pallas import tpu as pltpu

_SQRT1_2 = 0.7071067811865476


def _gelu(x):
    return 0.5 * x * (1.0 + jax.lax.erf(x * _SQRT1_2))


def _kl_term(mu1, mu2, ls1, ls2):
    return -0.5 + ls2 - ls1 + 0.5 * (
        jnp.exp(2.0 * (ls1 - ls2)) + (mu1 - mu2) ** 2 * jnp.exp(-2.0 * ls2))


def _dgt(w, v, prec=jnp.float32):
    return jax.lax.dot_general(w, v, (((0,), (0,)), ((), ())),
                               preferred_element_type=prec)


def _fwd_kernel(full_ref, part_ref, eps_ref,
                e0, e1, e2, e3, p0, p1, p2, p3, zw, rp_ref, bp_ref,
                z_ref, x_ref, kl_ref, klp_ref, klq_ref, *, zd, nb):
    c, hw = full_ref.shape[1:]
    fvec = jnp.concatenate(
        [jnp.mean(full_ref[i], axis=1, keepdims=True) for i in range(nb)],
        axis=1)
    pvec = jnp.concatenate(
        [jnp.mean(part_ref[i], axis=1, keepdims=True) for i in range(nb)],
        axis=1)

    v = fvec
    for w_ref, boff in ((e0, 0), (e1, c), (e2, 2 * c)):
        v = _dgt(w_ref[...], _gelu(v)) + bp_ref[boff:boff + c]
    ev = _dgt(e3[...], _gelu(v)) + bp_ref[3 * c:3 * c + 2 * zd]

    pb0 = 3 * c + 2 * zd
    u = pvec
    for w_ref, boff in ((p0, pb0), (p1, pb0 + c), (p2, pb0 + 2 * c)):
        u = _dgt(w_ref[...], _gelu(u)) + bp_ref[boff:boff + c]
    po = _dgt(p3[...], _gelu(u)) + bp_ref[pb0 + 3 * c:pb0 + 4 * c + 2 * zd]

    qm, qv = ev[0:zd], ev[zd:2 * zd]
    pm, pvr = po[0:zd], po[zd:2 * zd]
    xpp = po[2 * zd:]
    eps = jnp.transpose(eps_ref[...][:, :, 0])

    z = jnp.exp(qv) * eps + qm
    zb0 = 7 * c + 4 * zd
    xs = xpp + _dgt(zw[...], z) + bp_ref[zb0:zb0 + c]

    kl = _kl_term(qm, pm, qv, pvr)
    klq = _kl_term(qm, 0.0, qv, 0.0)
    klp = _kl_term(pm, 0.0, pvr, 0.0)
    z_ref[...] = jnp.transpose(z).reshape(nb, zd, 1, 1)
    kl_ref[...] = jnp.transpose(kl).reshape(nb, zd, 1, 1)
    klq_ref[...] = jnp.transpose(klq).reshape(nb, zd, 1, 1)
    klp_ref[...] = jnp.transpose(klp).reshape(nb, zd, 1, 1)

    rb0 = 8 * c + 4 * zd
    for i in range(nb):
        xin = full_ref[i] + xs[:, i:i + 1]
        hh = xin
        for l in range(4):
            hb = hh.astype(jnp.bfloat16)
            t = jax.lax.erf(hh * _SQRT1_2).astype(jnp.bfloat16)
            g = hb * (jnp.bfloat16(1.0) + t)
            hh = _dgt(rp_ref[:, l * c:(l + 1) * c], g) + bp_ref[
                rb0 + l * c:rb0 + (l + 1) * c]
        x_ref[i] = xin + hh


def kernel(full_acts, part_acts, eps,
           enc0_w, enc0_b, enc1_w, enc1_b, enc2_w, enc2_b, enc3_w, enc3_b,
           prior0_w, prior0_b, prior1_w, prior1_b, prior2_w, prior2_b,
           prior3_w, prior3_b,
           res0_w, res0_b, res1_w, res1_b, res2_w, res2_b, res3_w, res3_b,
           zp_w, zp_b):
    B, C, H, W = full_acts.shape
    HW = H * W
    zd = eps.shape[1]

    full2 = full_acts.reshape(B, C, HW)
    part2 = part_acts.reshape(B, C, HW)
    eps3 = eps[:, :, None]
    rpack = (0.5 * jnp.concatenate([res0_w, res1_w, res2_w, res3_w],
                                   axis=1)).astype(jnp.bfloat16)
    bpackt = jnp.concatenate(
        [enc0_b, enc1_b, enc2_b, enc3_b, prior0_b, prior1_b, prior2_b,
         prior3_b, zp_b, res0_b, res1_b, res2_b, res3_b], axis=1).T

    whole = lambda a: pl.BlockSpec(a.shape, lambda b: (0,) * a.ndim)
    nb = 4 if B % 4 == 0 else 1
    small_spec = pl.BlockSpec((nb, zd, 1, 1), lambda b: (b, 0, 0, 0))

    def run(f2, p2, e3_, ew0, ew1, ew2, ew3, pw0, pw1, pw2, pw3, zw, rp, bp):
        nloc = f2.shape[0]
        sm = jax.ShapeDtypeStruct((nloc, zd, 1, 1), jnp.float32)
        return pl.pallas_call(
            functools.partial(_fwd_kernel, zd=zd, nb=nb),
            grid=(nloc // nb,),
            in_specs=[pl.BlockSpec((nb, C, HW), lambda b: (b, 0, 0)),
                      pl.BlockSpec((nb, C, HW), lambda b: (b, 0, 0)),
                      pl.BlockSpec((nb, zd, 1), lambda b: (b, 0, 0)),
                      whole(ew0), whole(ew1), whole(ew2), whole(ew3),
                      whole(pw0), whole(pw1), whole(pw2), whole(pw3),
                      whole(zw), whole(rp), whole(bp)],
            out_specs=(small_spec,
                       pl.BlockSpec((nb, C, HW), lambda b: (b, 0, 0)),
                       small_spec, small_spec, small_spec),
            out_shape=(sm,
                       jax.ShapeDtypeStruct((nloc, C, HW), jnp.float32),
                       sm, sm, sm),
            compiler_params=pltpu.CompilerParams(
                dimension_semantics=("parallel",),
                vmem_limit_bytes=60 * 1024 * 1024),
        )(f2, p2, e3_, ew0, ew1, ew2, ew3, pw0, pw1, pw2, pw3, zw, rp, bp)

    z4, xd, kl4, klp4, klq4 = run(
        full2, part2, eps3, enc0_w, enc1_w, enc2_w, enc3_w,
        prior0_w, prior1_w, prior2_w, prior3_w, zp_w, rpack, bpackt)

    x = xd.reshape(B, C, H, W)
    return z4, x, kl4, klp4, klq4
